# baseline (speedup 1.0000x reference)
import numpy as np
import jax
import jax.numpy as jnp
from jax.experimental import pallas as pl

NUM_BANDS = 64
MAP_FREQ = 200
SP = 10
ROWS = 257 * SP
RECS = 128 * SP

_SIN_C = (3.1415927, -5.167711, 2.550092, -0.5983952, 0.07788843)


def _sinpi(t):
    n = jnp.round(t)
    r = t - n
    sgn = jax.lax.shift_left(n.astype(jnp.int32), 31)
    s = r * r
    sp = _SIN_C[4]
    for i in (3, 2, 1, 0):
        sp = sp * s + _SIN_C[i]
    sp = sp * r
    return jax.lax.bitcast_convert_type(
        jax.lax.bitcast_convert_type(sp, jnp.int32) ^ sgn, jnp.float32
    )


def _tables():
    q = np.arange(257)[:, None]
    l = np.arange(128)[None, :]
    e = 128 * q + l
    c = e % 257
    ro = e // 257
    n0 = (128 * q) // 257
    freqs = np.linspace(1.0, MAP_FREQ / 2.0, NUM_BANDS).astype(np.float32)
    mx1 = ((c % 128) >= 64) & (c < 256)
    mn = ro > n0
    mpass = c == 256
    fph = np.where(c < 256, freqs[c % NUM_BANDS], 0.0).astype(np.float32)
    ph = np.where((c >= 128) & (c < 256), 0.5, 0.0).astype(np.float32)
    i = np.arange(128)[None, :]
    p2a = (i == n0).astype(np.float32)
    p2b = (i == np.minimum(n0 + 1, 127)).astype(np.float32)
    return mx1, mn, mpass, fph, ph, p2a, p2b


def _enc_kernel(ga_ref, gb_ref, mx1_ref, mn_ref, mpass_ref,
                fph_ref, ph_ref, out_ref):
    mx1 = mx1_ref[...] != 0
    mn = mn_ref[...] != 0
    mpass = mpass_ref[...] != 0
    fph = fph_ref[...]
    ph = ph_ref[...]
    for s in range(SP):
        g = ga_ref[s]
        gb = gb_ref[s]
        a = jnp.where(mx1, g[:, 1:2], g[:, 0:1])
        bn = jnp.where(mx1, gb[:, 1:2], gb[:, 0:1])
        x01 = jnp.where(mn, bn, a)
        t = x01 * fph + ph
        v = _sinpi(t)
        x2v = jnp.where(mn, gb[:, 2:3], g[:, 2:3])
        v = jnp.where(mpass, jnp.broadcast_to(x2v, v.shape), v)
        vv = jnp.where(mn, gb[:, 3:4], g[:, 3:4])
        v = v * vv
        out_ref[s, :, :] = v


def kernel(x, pad_mask):
    B, N, _ = x.shape
    C = 4 * NUM_BANDS + 1
    P = B * (N + 1)
    total_rows = P * C // 128
    grid = total_rows // ROWS

    xr = jnp.concatenate([x, jnp.zeros((B, 1, 3), x.dtype)], axis=1)
    comp = xr.reshape(P, 3)
    valid = ((jnp.arange(P) % (N + 1)) != N).astype(x.dtype)
    rec = jnp.concatenate([comp, valid[:, None]], axis=1)

    mx1, mn, mpass, fph, ph, p2a, p2b = _tables()
    nsp = total_rows // 257
    n0 = (128 * np.arange(257)) // 257
    idx_a = 128 * np.arange(nsp)[:, None] + n0[None, :]
    idx_b = np.minimum(idx_a + 1, P - 1)
    ga = rec[jnp.asarray(idx_a)]
    gb = rec[jnp.asarray(idx_b)]
    ci = lambda g: (0, 0)

    flat = pl.pallas_call(
        _enc_kernel,
        grid=(grid,),
        in_specs=[
            pl.BlockSpec((SP, 257, 4), lambda g: (g, 0, 0)),
            pl.BlockSpec((SP, 257, 4), lambda g: (g, 0, 0)),
            pl.BlockSpec((257, 128), ci),
            pl.BlockSpec((257, 128), ci),
            pl.BlockSpec((257, 128), ci),
            pl.BlockSpec((257, 128), ci),
            pl.BlockSpec((257, 128), ci),
        ],
        out_specs=pl.BlockSpec((SP, 257, 128), lambda g: (g, 0, 0)),
        out_shape=jax.ShapeDtypeStruct((total_rows // 257, 257, 128), x.dtype),
    )(
        ga, gb,
        jnp.asarray(mx1.astype(np.int32)), jnp.asarray(mn.astype(np.int32)),
        jnp.asarray(mpass.astype(np.int32)),
        jnp.asarray(fph), jnp.asarray(ph),
    )

    enc = flat.reshape(B, N + 1, C)
    out_mask = jnp.concatenate(
        [pad_mask, jnp.zeros((B, 1), dtype=pad_mask.dtype)], axis=1
    )
    return (enc, out_mask)
